# baseline (speedup 1.0000x reference)
import functools

import jax
import jax.numpy as jnp
from jax import lax
from jax.experimental import pallas as pl
from jax.experimental.pallas import tpu as pltpu
from jax.experimental.pallas import tpu_sc as plsc

D_FEAT = 128
D_EDGE = 16
OUT_DIM = 128
_SC_BLOCK = 128


def _nodeproj_body(nf, ws, wd, ps, pd):
    x = nf[...]
    ps[...] = jnp.dot(x, ws[...], preferred_element_type=jnp.float32)
    pd[...] = jnp.dot(x, wd[...], preferred_element_type=jnp.float32)


def _node_projections(node_feat, W_s, W_d):
    N = node_feat.shape[0]
    BLK = 2000
    return pl.pallas_call(
        _nodeproj_body,
        grid=(N // BLK,),
        in_specs=[
            pl.BlockSpec((BLK, D_FEAT), lambda i: (i, 0)),
            pl.BlockSpec((D_FEAT, OUT_DIM), lambda i: (0, 0)),
            pl.BlockSpec((D_FEAT, OUT_DIM), lambda i: (0, 0)),
        ],
        out_specs=[
            pl.BlockSpec((BLK, OUT_DIM), lambda i: (i, 0)),
            pl.BlockSpec((BLK, OUT_DIM), lambda i: (i, 0)),
        ],
        out_shape=[
            jax.ShapeDtypeStruct((N, OUT_DIM), jnp.float32),
            jax.ShapeDtypeStruct((N, OUT_DIM), jnp.float32),
        ],
    )(node_feat, W_s, W_d)


def _make_gather_sum(E):
    info = plsc.get_sparse_core_info()
    NC, NS = info.num_cores, info.num_subcores
    NW = NC * NS
    B = _SC_BLOCK
    nblk = E // B
    mesh = plsc.VectorSubcoreMesh(core_axis_name="c", subcore_axis_name="s")

    Q, R = divmod(nblk, 32)
    NMAX = Q + (1 if R else 0)

    PK = OUT_DIM // 2

    @functools.partial(
        pl.kernel,
        mesh=mesh,
        compiler_params=pltpu.CompilerParams(use_tc_tiling_on_sc=False),
        out_type=jax.ShapeDtypeStruct((E, OUT_DIM), jnp.float32),
        scratch_types=[
            pltpu.VMEM((NMAX * B,), jnp.int32),
            pltpu.VMEM((NMAX * B,), jnp.int32),
            pltpu.VMEM((B, PK), jnp.int32),
            pltpu.VMEM((B, PK), jnp.int32),
            pltpu.VMEM((B, PK), jnp.int32),
            pltpu.VMEM((B, PK), jnp.int32),
            pltpu.VMEM((B, OUT_DIM), jnp.float32),
            pltpu.VMEM((B, OUT_DIM), jnp.float32),
            pltpu.SemaphoreType.DMA,
            pltpu.SemaphoreType.DMA,
            pltpu.SemaphoreType.DMA,
            pltpu.SemaphoreType.DMA,
        ],
    )
    def gather_sum(ps_hbm, pd_hbm, src_hbm, dst_hbm, g_hbm,
                   idx_s, idx_d, buf_s0, buf_s1, buf_d0, buf_d1,
                   buf_o0, buf_o1, sem_g0, sem_g1, sem_w0, sem_w1):
        wid = lax.axis_index("s") * NC + lax.axis_index("c")
        my_n = Q + jnp.where(wid < R, 1, 0)
        start = wid * Q + jnp.minimum(wid, R)
        estart = start * B
        pltpu.sync_copy(src_hbm.at[pl.ds(estart, Q * B)],
                        idx_s.at[pl.ds(0, Q * B)])
        pltpu.sync_copy(dst_hbm.at[pl.ds(estart, Q * B)],
                        idx_d.at[pl.ds(0, Q * B)])

        @pl.when(my_n > Q)
        def _():
            pltpu.sync_copy(src_hbm.at[pl.ds(estart + Q * B, B)],
                            idx_s.at[pl.ds(Q * B, B)])
            pltpu.sync_copy(dst_hbm.at[pl.ds(estart + Q * B, B)],
                            idx_d.at[pl.ds(Q * B, B)])

        bufs = ((buf_s0, buf_d0, buf_o0, sem_g0, sem_w0),
                (buf_s1, buf_d1, buf_o1, sem_g1, sem_w1))

        def issue(slot, i, guard):
            bs, bd, _, sg, _ = bufs[slot]

            def _go():
                pltpu.async_copy(ps_hbm.at[idx_s.at[pl.ds(i * B, B)]], bs, sg)
                pltpu.async_copy(pd_hbm.at[idx_d.at[pl.ds(i * B, B)]], bd, sg)

            if guard:
                pl.when(i < my_n)(_go)
            else:
                _go()

        def finish(slot, i, wait_prev_wb):
            bs, bd, bo, sg, sw = bufs[slot]

            @pl.when(i < my_n)
            def _():
                base = (start + i) * B
                pltpu.make_async_copy(ps_hbm.at[pl.ds(0, B)], bs, sg).wait()
                pltpu.make_async_copy(pd_hbm.at[pl.ds(0, B)], bd, sg).wait()
                hi_mask = jnp.int32(-65536)
                if wait_prev_wb:
                    pltpu.make_async_copy(bo, g_hbm.at[pl.ds(0, B)],
                                          sw).wait()

                def row_body(r, rcarry):
                    for c in range(PK // 16):
                        sl = pl.ds(c * 16, 16)
                        sv = bs[r, sl]
                        dv = bd[r, sl]
                        fl = (lax.bitcast_convert_type(sv << 16, jnp.float32)
                              + lax.bitcast_convert_type(dv << 16,
                                                         jnp.float32))
                        fh = (lax.bitcast_convert_type(sv & hi_mask,
                                                       jnp.float32)
                              + lax.bitcast_convert_type(dv & hi_mask,
                                                         jnp.float32))
                        bo[r, pl.ds(c * 16, 16)] = fl
                        bo[r, pl.ds(c * 16 + PK, 16)] = fh
                    return rcarry

                lax.fori_loop(0, B, row_body, 0)
                pltpu.async_copy(bo, g_hbm.at[pl.ds(base, B)], sw)

        issue(0, 0, guard=False)
        issue(1, 1, guard=False)
        finish(0, 0, wait_prev_wb=False)
        issue(0, 2, guard=True)
        finish(1, 1, wait_prev_wb=False)
        issue(1, 3, guard=True)

        def pair_body(p, carry):
            i0 = p * 2
            finish(0, i0, wait_prev_wb=True)
            issue(0, i0 + 2, guard=True)
            finish(1, i0 + 1, wait_prev_wb=True)
            issue(1, i0 + 3, guard=True)
            return carry

        lax.fori_loop(1, (NMAX + 1) // 2 + 1, pair_body, 0)

    return gather_sum


def _edge_body(ea, we, bb, g, out):
    out[...] = (g[...]
                + jnp.dot(ea[...], we[...], preferred_element_type=jnp.float32)
                + bb[...])


def _edge_combine(edge_attr, W_e, b2d, G):
    E = edge_attr.shape[0]
    BLK = 4000
    return pl.pallas_call(
        _edge_body,
        grid=(E // BLK,),
        in_specs=[
            pl.BlockSpec((BLK, D_EDGE), lambda i: (i, 0)),
            pl.BlockSpec((D_EDGE, OUT_DIM), lambda i: (0, 0)),
            pl.BlockSpec((1, OUT_DIM), lambda i: (0, 0)),
            pl.BlockSpec((BLK, OUT_DIM), lambda i: (i, 0)),
        ],
        out_specs=pl.BlockSpec((BLK, OUT_DIM), lambda i: (i, 0)),
        out_shape=jax.ShapeDtypeStruct((E, OUT_DIM), jnp.float32),
    )(edge_attr, W_e, b2d, G)


def _pack_cols(p):
    h = OUT_DIM // 2
    pb = p.astype(jnp.bfloat16)
    return jax.lax.bitcast_convert_type(
        jnp.stack([pb[:, :h], pb[:, h:]], axis=-1), jnp.int32)


def kernel(node_feat, edge_attr, edge_index, W, b):
    W_e = W[:D_EDGE]
    W_s = W[D_EDGE:D_EDGE + D_FEAT]
    W_d = W[D_EDGE + D_FEAT:]
    src = edge_index[0]
    dst = edge_index[1]
    ps, pd = _node_projections(node_feat, W_s, W_d)
    G = _make_gather_sum(edge_attr.shape[0])(_pack_cols(ps), _pack_cols(pd),
                                             src, dst)
    return _edge_combine(edge_attr, W_e, b.reshape(1, OUT_DIM), G)


# speedup vs baseline: 1.1082x; 1.1082x over previous
import functools

import jax
import jax.numpy as jnp
from jax import lax
from jax.experimental import pallas as pl
from jax.experimental.pallas import tpu as pltpu
from jax.experimental.pallas import tpu_sc as plsc

D_FEAT = 128
D_EDGE = 16
OUT_DIM = 128
_SC_BLOCK = 128


def _nodeproj_body(nf, ws, wd, ps, pd):
    x = nf[...]
    ps[...] = jnp.dot(x, ws[...], preferred_element_type=jnp.float32)
    pd[...] = jnp.dot(x, wd[...], preferred_element_type=jnp.float32)


def _node_projections(node_feat, W_s, W_d):
    N = node_feat.shape[0]
    BLK = 2000
    return pl.pallas_call(
        _nodeproj_body,
        grid=(N // BLK,),
        in_specs=[
            pl.BlockSpec((BLK, D_FEAT), lambda i: (i, 0)),
            pl.BlockSpec((D_FEAT, OUT_DIM), lambda i: (0, 0)),
            pl.BlockSpec((D_FEAT, OUT_DIM), lambda i: (0, 0)),
        ],
        out_specs=[
            pl.BlockSpec((BLK, OUT_DIM), lambda i: (i, 0)),
            pl.BlockSpec((BLK, OUT_DIM), lambda i: (i, 0)),
        ],
        out_shape=[
            jax.ShapeDtypeStruct((N, OUT_DIM), jnp.float32),
            jax.ShapeDtypeStruct((N, OUT_DIM), jnp.float32),
        ],
    )(node_feat, W_s, W_d)


def _make_gather_sum(E):
    info = plsc.get_sparse_core_info()
    NC, NS = info.num_cores, info.num_subcores
    NW = NC * NS
    B = _SC_BLOCK
    nblk = E // B
    mesh = plsc.VectorSubcoreMesh(core_axis_name="c", subcore_axis_name="s")

    Q, R = divmod(nblk, 32)
    NMAX = Q + (1 if R else 0)

    @functools.partial(
        pl.kernel,
        mesh=mesh,
        out_type=jax.ShapeDtypeStruct((E, OUT_DIM), jnp.float32),
        scratch_types=[
            pltpu.VMEM((NMAX * B,), jnp.int32),
            pltpu.VMEM((NMAX * B,), jnp.int32),
            pltpu.VMEM((B, OUT_DIM), jnp.float32),
            pltpu.VMEM((B, OUT_DIM), jnp.float32),
            pltpu.VMEM((B, OUT_DIM), jnp.float32),
            pltpu.VMEM((B, OUT_DIM), jnp.float32),
            pltpu.VMEM((B, OUT_DIM), jnp.float32),
            pltpu.VMEM((B, OUT_DIM), jnp.float32),
            pltpu.SemaphoreType.DMA,
            pltpu.SemaphoreType.DMA,
            pltpu.SemaphoreType.DMA,
            pltpu.SemaphoreType.DMA,
        ],
    )
    def gather_sum(ps_hbm, pd_hbm, src_hbm, dst_hbm, g_hbm,
                   idx_s, idx_d, buf_s0, buf_s1, buf_d0, buf_d1,
                   buf_o0, buf_o1, sem_g0, sem_g1, sem_w0, sem_w1):
        wid = lax.axis_index("s") * NC + lax.axis_index("c")
        my_n = Q + jnp.where(wid < R, 1, 0)
        start = wid * Q + jnp.minimum(wid, R)
        estart = start * B
        pltpu.sync_copy(src_hbm.at[pl.ds(estart, Q * B)],
                        idx_s.at[pl.ds(0, Q * B)])
        pltpu.sync_copy(dst_hbm.at[pl.ds(estart, Q * B)],
                        idx_d.at[pl.ds(0, Q * B)])

        @pl.when(my_n > Q)
        def _():
            pltpu.sync_copy(src_hbm.at[pl.ds(estart + Q * B, B)],
                            idx_s.at[pl.ds(Q * B, B)])
            pltpu.sync_copy(dst_hbm.at[pl.ds(estart + Q * B, B)],
                            idx_d.at[pl.ds(Q * B, B)])

        bufs = ((buf_s0, buf_d0, buf_o0, sem_g0, sem_w0),
                (buf_s1, buf_d1, buf_o1, sem_g1, sem_w1))

        def issue(slot, i, guard):
            bs, bd, _, sg, _ = bufs[slot]

            def _go():
                pltpu.async_copy(ps_hbm.at[idx_s.at[pl.ds(i * B, B)]], bs, sg)
                pltpu.async_copy(pd_hbm.at[idx_d.at[pl.ds(i * B, B)]], bd, sg)

            if guard:
                pl.when(i < my_n)(_go)
            else:
                _go()

        def finish(slot, i, wait_prev_wb):
            bs, bd, bo, sg, sw = bufs[slot]

            @pl.when(i < my_n)
            def _():
                base = (start + i) * B
                pltpu.make_async_copy(ps_hbm.at[pl.ds(0, B)], bs, sg).wait()
                pltpu.make_async_copy(pd_hbm.at[pl.ds(0, B)], bd, sg).wait()
                if wait_prev_wb:
                    pltpu.make_async_copy(bo, g_hbm.at[pl.ds(0, B)],
                                          sw).wait()

                def row_body(r, rcarry):
                    for c in range(OUT_DIM // 16):
                        sl = pl.ds(c * 16, 16)
                        bo[r, sl] = bs[r, sl] + bd[r, sl]
                    return rcarry

                lax.fori_loop(0, B, row_body, 0)
                pltpu.async_copy(bo, g_hbm.at[pl.ds(base, B)], sw)

        issue(0, 0, guard=False)
        issue(1, 1, guard=False)
        finish(0, 0, wait_prev_wb=False)
        issue(0, 2, guard=True)
        finish(1, 1, wait_prev_wb=False)
        issue(1, 3, guard=True)

        def pair_body(p, carry):
            i0 = p * 2
            finish(0, i0, wait_prev_wb=True)
            issue(0, i0 + 2, guard=True)
            finish(1, i0 + 1, wait_prev_wb=True)
            issue(1, i0 + 3, guard=True)
            return carry

        lax.fori_loop(1, (NMAX + 1) // 2 + 1, pair_body, 0)

    return gather_sum


def _edge_body(ea, we, bb, g, out):
    out[...] = (g[...]
                + jnp.dot(ea[...], we[...], preferred_element_type=jnp.float32)
                + bb[...])


def _edge_body_chain(ea, we, bb, g, prev, out):
    del prev
    out[...] = (g[...]
                + jnp.dot(ea[...], we[...], preferred_element_type=jnp.float32)
                + bb[...])


def _edge_combine_chunk(edge_attr_c, W_e, b2d, G_c, chunk, E, prev):
    EC = edge_attr_c.shape[0]
    BLK = 4000
    off = chunk * (EC // BLK)
    in_specs = [
        pl.BlockSpec((BLK, D_EDGE), lambda i: (i, 0)),
        pl.BlockSpec((D_EDGE, OUT_DIM), lambda i: (0, 0)),
        pl.BlockSpec((1, OUT_DIM), lambda i: (0, 0)),
        pl.BlockSpec((BLK, OUT_DIM), lambda i: (i, 0)),
    ]
    args = [edge_attr_c, W_e, b2d, G_c]
    if prev is None:
        body = _edge_body
        aliases = {}
    else:
        body = _edge_body_chain
        in_specs.append(pl.BlockSpec((8, OUT_DIM), lambda i: (0, 0)))
        args.append(prev)
        aliases = {4: 0}
    return pl.pallas_call(
        body,
        grid=(EC // BLK,),
        in_specs=in_specs,
        out_specs=pl.BlockSpec((BLK, OUT_DIM), lambda i: (i + off, 0)),
        out_shape=jax.ShapeDtypeStruct((E, OUT_DIM), jnp.float32),
        input_output_aliases=aliases,
    )(*args)


def kernel(node_feat, edge_attr, edge_index, W, b):
    W_e = W[:D_EDGE]
    W_s = W[D_EDGE:D_EDGE + D_FEAT]
    W_d = W[D_EDGE + D_FEAT:]
    src = edge_index[0]
    dst = edge_index[1]
    ps, pd = _node_projections(node_feat, W_s, W_d)
    E = edge_attr.shape[0]
    C = 4
    EC = E // C
    b2d = b.reshape(1, OUT_DIM)
    sc_call = _make_gather_sum(EC)
    out = None
    for c in range(C):
        lo, hi = c * EC, (c + 1) * EC
        G_c = sc_call(ps, pd, src[lo:hi], dst[lo:hi])
        out = _edge_combine_chunk(edge_attr[lo:hi], W_e, b2d, G_c, c, E, out)
    return out


# speedup vs baseline: 1.1394x; 1.0282x over previous
import functools

import jax
import jax.numpy as jnp
from jax import lax
from jax.experimental import pallas as pl
from jax.experimental.pallas import tpu as pltpu
from jax.experimental.pallas import tpu_sc as plsc

D_FEAT = 128
D_EDGE = 16
OUT_DIM = 128
_SC_BLOCK = 128


def _nodeproj_body(nf, ws, wd, ps, pd):
    x = nf[...]
    ps[...] = jnp.dot(x, ws[...], preferred_element_type=jnp.float32)
    pd[...] = jnp.dot(x, wd[...], preferred_element_type=jnp.float32)


def _node_projections(node_feat, W_s, W_d):
    N = node_feat.shape[0]
    BLK = 2000
    return pl.pallas_call(
        _nodeproj_body,
        grid=(N // BLK,),
        in_specs=[
            pl.BlockSpec((BLK, D_FEAT), lambda i: (i, 0)),
            pl.BlockSpec((D_FEAT, OUT_DIM), lambda i: (0, 0)),
            pl.BlockSpec((D_FEAT, OUT_DIM), lambda i: (0, 0)),
        ],
        out_specs=[
            pl.BlockSpec((BLK, OUT_DIM), lambda i: (i, 0)),
            pl.BlockSpec((BLK, OUT_DIM), lambda i: (i, 0)),
        ],
        out_shape=[
            jax.ShapeDtypeStruct((N, OUT_DIM), jnp.float32),
            jax.ShapeDtypeStruct((N, OUT_DIM), jnp.float32),
        ],
    )(node_feat, W_s, W_d)


def _make_gather_sum(E):
    info = plsc.get_sparse_core_info()
    NC, NS = info.num_cores, info.num_subcores
    NW = NC * NS
    B = _SC_BLOCK
    nblk = E // B
    mesh = plsc.VectorSubcoreMesh(core_axis_name="c", subcore_axis_name="s")

    Q, R = divmod(nblk, 32)
    NMAX = Q + (1 if R else 0)

    @functools.partial(
        pl.kernel,
        mesh=mesh,
        out_type=jax.ShapeDtypeStruct((E, OUT_DIM), jnp.float32),
        scratch_types=[
            pltpu.VMEM((NMAX * B,), jnp.int32),
            pltpu.VMEM((NMAX * B,), jnp.int32),
            pltpu.VMEM((B, OUT_DIM), jnp.float32),
            pltpu.VMEM((B, OUT_DIM), jnp.float32),
            pltpu.VMEM((B, OUT_DIM), jnp.float32),
            pltpu.VMEM((B, OUT_DIM), jnp.float32),
            pltpu.VMEM((B, OUT_DIM), jnp.float32),
            pltpu.VMEM((B, OUT_DIM), jnp.float32),
            pltpu.SemaphoreType.DMA,
            pltpu.SemaphoreType.DMA,
            pltpu.SemaphoreType.DMA,
            pltpu.SemaphoreType.DMA,
        ],
    )
    def gather_sum(ps_hbm, pd_hbm, src_hbm, dst_hbm, g_hbm,
                   idx_s, idx_d, buf_s0, buf_s1, buf_d0, buf_d1,
                   buf_o0, buf_o1, sem_g0, sem_g1, sem_w0, sem_w1):
        wid = lax.axis_index("s") * NC + lax.axis_index("c")
        my_n = Q + jnp.where(wid < R, 1, 0)
        start = wid * Q + jnp.minimum(wid, R)
        estart = start * B
        pltpu.sync_copy(src_hbm.at[pl.ds(estart, Q * B)],
                        idx_s.at[pl.ds(0, Q * B)])
        pltpu.sync_copy(dst_hbm.at[pl.ds(estart, Q * B)],
                        idx_d.at[pl.ds(0, Q * B)])

        @pl.when(my_n > Q)
        def _():
            pltpu.sync_copy(src_hbm.at[pl.ds(estart + Q * B, B)],
                            idx_s.at[pl.ds(Q * B, B)])
            pltpu.sync_copy(dst_hbm.at[pl.ds(estart + Q * B, B)],
                            idx_d.at[pl.ds(Q * B, B)])

        bufs = ((buf_s0, buf_d0, buf_o0, sem_g0, sem_w0),
                (buf_s1, buf_d1, buf_o1, sem_g1, sem_w1))

        def issue(slot, i, guard):
            bs, bd, _, sg, _ = bufs[slot]

            def _go():
                pltpu.async_copy(ps_hbm.at[idx_s.at[pl.ds(i * B, B)]], bs, sg)
                pltpu.async_copy(pd_hbm.at[idx_d.at[pl.ds(i * B, B)]], bd, sg)

            if guard:
                pl.when(i < my_n)(_go)
            else:
                _go()

        def finish(slot, i, wait_prev_wb):
            bs, bd, bo, sg, sw = bufs[slot]

            @pl.when(i < my_n)
            def _():
                base = (start + i) * B
                pltpu.make_async_copy(ps_hbm.at[pl.ds(0, B)], bs, sg).wait()
                pltpu.make_async_copy(pd_hbm.at[pl.ds(0, B)], bd, sg).wait()
                if wait_prev_wb:
                    pltpu.make_async_copy(bo, g_hbm.at[pl.ds(0, B)],
                                          sw).wait()

                def row_body(r, rcarry):
                    for c in range(OUT_DIM // 16):
                        sl = pl.ds(c * 16, 16)
                        bo[r, sl] = bs[r, sl] + bd[r, sl]
                    return rcarry

                lax.fori_loop(0, B, row_body, 0)
                pltpu.async_copy(bo, g_hbm.at[pl.ds(base, B)], sw)

        issue(0, 0, guard=False)
        issue(1, 1, guard=False)
        finish(0, 0, wait_prev_wb=False)
        issue(0, 2, guard=True)
        finish(1, 1, wait_prev_wb=False)
        issue(1, 3, guard=True)

        def pair_body(p, carry):
            i0 = p * 2
            finish(0, i0, wait_prev_wb=True)
            issue(0, i0 + 2, guard=True)
            finish(1, i0 + 1, wait_prev_wb=True)
            issue(1, i0 + 3, guard=True)
            return carry

        lax.fori_loop(1, (NMAX + 1) // 2 + 1, pair_body, 0)

    return gather_sum


def _edge_body(ea, we, bb, g, out):
    out[...] = (g[...]
                + jnp.dot(ea[...], we[...], preferred_element_type=jnp.float32)
                + bb[...])


def _edge_combine(edge_attr, W_e, b2d, G):
    E = edge_attr.shape[0]
    BLK = 8000
    return pl.pallas_call(
        _edge_body,
        grid=(E // BLK,),
        in_specs=[
            pl.BlockSpec((BLK, D_EDGE), lambda i: (i, 0)),
            pl.BlockSpec((D_EDGE, OUT_DIM), lambda i: (0, 0)),
            pl.BlockSpec((1, OUT_DIM), lambda i: (0, 0)),
            pl.BlockSpec((BLK, OUT_DIM), lambda i: (i, 0)),
        ],
        out_specs=pl.BlockSpec((BLK, OUT_DIM), lambda i: (i, 0)),
        out_shape=jax.ShapeDtypeStruct((E, OUT_DIM), jnp.float32),
    )(edge_attr, W_e, b2d, G)


def kernel(node_feat, edge_attr, edge_index, W, b):
    W_e = W[:D_EDGE]
    W_s = W[D_EDGE:D_EDGE + D_FEAT]
    W_d = W[D_EDGE + D_FEAT:]
    src = edge_index[0]
    dst = edge_index[1]
    ps, pd = _node_projections(node_feat, W_s, W_d)
    G = _make_gather_sum(edge_attr.shape[0])(ps, pd, src, dst)
    return _edge_combine(edge_attr, W_e, b.reshape(1, OUT_DIM), G)
